# baseline (speedup 1.0000x reference)
import functools

import jax
import jax.numpy as jnp
from jax import lax
from jax.experimental import pallas as pl
from jax.experimental.pallas import tpu as pltpu
from jax.experimental.pallas import tpu_sc as plsc

_L16 = 16
_NC = 2
_NS = 16
_NW = _NC * _NS
_REP = 8


def _make_sc_kernel(batch, length, d):
    rows_per_w = batch // _NW
    ndma = rows_per_w // _REP
    nvec_per_row = (length * d) // _L16
    mesh = plsc.VectorSubcoreMesh(core_axis_name="c", subcore_axis_name="s")

    @functools.partial(
        pl.kernel,
        mesh=mesh,
        compiler_params=pltpu.CompilerParams(use_tc_tiling_on_sc=False),
        out_type=jax.ShapeDtypeStruct((batch, length, d), jnp.float32),
        scratch_types=[
            pltpu.VMEM((length, d), jnp.float32),
            pltpu.VMEM((length, d), jnp.float32),
            pltpu.VMEM((_REP, length, d), jnp.float32),
            pltpu.SemaphoreType.DMA,
        ],
    )
    def sc_kernel(emb_hbm, pos_hbm, out_hbm, emb_v, pos_v, rep_v, sem):
        wid = lax.axis_index("s") * _NC + lax.axis_index("c")
        base = wid * rows_per_w
        pltpu.sync_copy(emb_hbm, emb_v)
        pltpu.sync_copy(pos_hbm, pos_v)
        nv_row = d // _L16

        def add_body(i, _):
            p = i // nv_row
            q = i - p * nv_row
            sl = pl.ds(q * _L16, _L16)
            v = emb_v[p, sl] + pos_v[p, sl]
            for r in range(_REP):
                rep_v[r, p, sl] = v
            return 0

        lax.fori_loop(0, nvec_per_row, add_body, 0)

        copies = [
            pltpu.make_async_copy(
                rep_v, out_hbm.at[pl.ds(base + j * _REP, _REP)], sem
            )
            for j in range(ndma)
        ]
        for c in copies:
            c.start()
        for c in copies:
            c.wait()

    return sc_kernel


def kernel(input_char, emb_table, pos_table):
    batch, length = input_char.shape
    d = emb_table.shape[1]
    sc_kernel = _make_sc_kernel(batch, length, d)
    return sc_kernel(emb_table[:length], pos_table[0])
